# baseline (speedup 1.0000x reference)
import functools

import jax
import jax.numpy as jnp
from jax import lax
from jax.experimental import pallas as pl
from jax.experimental.pallas import tpu as pltpu
from jax.experimental.pallas import tpu_sc as plsc

_NC = 2
_NS = 16
_NW = _NC * _NS

_PITCH = 129


@functools.lru_cache(maxsize=None)
def _build(n_chunks, D, n_out):
    cpw = n_chunks // _NW

    mesh = plsc.VectorSubcoreMesh(core_axis_name="c", subcore_axis_name="s")

    @functools.partial(
        pl.kernel,
        mesh=mesh,
        out_type=jax.ShapeDtypeStruct((n_out // 128, 128), jnp.float32),
        scratch_types=[
            pltpu.VMEM((cpw, 128), jnp.int32),
            pltpu.VMEM((2, 128, D), jnp.float32),
            pltpu.VMEM((2, D, _PITCH), jnp.float32),
            pltpu.SemaphoreType.DMA,
            pltpu.SemaphoreType.DMA,
            pltpu.SemaphoreType.DMA,
            pltpu.SemaphoreType.DMA,
        ],
        compiler_params=pltpu.CompilerParams(
            use_tc_tiling_on_sc=False, needs_layout_passes=False),
    )
    def sc_gather(table_hbm, idx_hbm, out_hbm, idx_v, g_v, t_v,
                  g0sem, g1sem, s0sem, s1sem):
        wid = lax.axis_index("s") * _NC + lax.axis_index("c")
        chunk0 = wid * cpw
        pltpu.sync_copy(idx_hbm.at[pl.ds(chunk0, cpw)], idx_v)

        iota16 = lax.iota(jnp.int32, 16)
        gsems = (g0sem, g1sem)
        ssems = (s0sem, s1sem)

        def gather(g, buf):
            pltpu.async_copy(table_hbm.at[idx_v.at[g]], g_v.at[buf],
                             gsems[buf])

        def wait_gather(g, buf):
            pltpu.make_async_copy(table_hbm.at[idx_v.at[g]], g_v.at[buf],
                                  gsems[buf]).wait()

        def transpose(buf):
            g_ref = g_v.at[buf]
            t_ref = t_v.at[buf]

            @plsc.parallel_loop(0, 128, 1, unroll=8)
            def tok(j):
                cols = iota16 * 0 + j
                for c0 in range(0, D, 16):
                    v = g_ref[j, pl.ds(c0, 16)]
                    plsc.store_scatter(t_ref, [iota16 + c0, cols], v)

        def store(g, buf):
            rho = chunk0 + g
            d1 = rho // 128
            e = rho % 128
            base = d1 * (128 * D) + e * 8
            for a in range(D // 8):
                pltpu.async_copy(
                    t_v.at[buf, pl.ds(a * 8, 8), pl.ds(0, 128)],
                    out_hbm.at[pl.ds(base + a * 1024, 8)],
                    ssems[buf])

        def drain_store(buf):
            for a in range(D // 8):
                pltpu.make_async_copy(
                    t_v.at[buf, pl.ds(a * 8, 8), pl.ds(0, 128)],
                    out_hbm.at[pl.ds(a * 1024, 8)],
                    ssems[buf]).wait()

        def half(p, g, buf):
            wait_gather(g, buf)
            pl.when(p > 0)(lambda: drain_store(buf))
            transpose(buf)
            store(g, buf)
            @pl.when(g + 2 < cpw)
            def _():
                gather(g + 2, buf)

        gather(0, 0)
        gather(1, 1)

        def pair(p, carry):
            g0 = p * 2
            half(p, g0, 0)
            half(p, g0 + 1, 1)
            return carry

        lax.fori_loop(0, cpw // 2, pair, 0)
        drain_store(0)
        drain_store(1)

    return sc_gather


_DEPAD_R = 4000


def _depad_block(i_ref, o_ref):
    x = i_ref[...]
    x3 = x.reshape(_DEPAD_R // 2, 2, 64)
    o_ref[...] = jnp.concatenate([x3[:, 0, :], x3[:, 1, :]], axis=1)


def _depad(t):
    V = t.shape[0]
    return pl.pallas_call(
        _depad_block,
        grid=(V // _DEPAD_R,),
        in_specs=[pl.BlockSpec((_DEPAD_R, 64), lambda i: (i, 0))],
        out_specs=pl.BlockSpec((_DEPAD_R // 2, 128), lambda i: (i, 0)),
        out_shape=jax.ShapeDtypeStruct((V // 2, 128), jnp.float32),
    )(t)


def kernel(x, tok_embed):
    s0, s1 = x.shape
    V, D = tok_embed.shape
    n_chunks = s0 * s1 // 128
    idx_t = x.T.reshape(n_chunks, 128).astype(jnp.int32)
    n_out = s0 * s1 * D
    table_lin = _depad(tok_embed).reshape(V, D)
    o = _build(n_chunks, D, n_out)(table_lin, idx_t)
    o5 = o.reshape(s1, D // 8, s0 // 128, 8, 128)
    return o5.transpose(2, 4, 0, 1, 3).reshape(s0, s1, D)


# speedup vs baseline: 1.2175x; 1.2175x over previous
import functools

import jax
import jax.numpy as jnp
from jax import lax
from jax.experimental import pallas as pl
from jax.experimental.pallas import tpu as pltpu
from jax.experimental.pallas import tpu_sc as plsc

_NC = 2
_NS = 16
_NW = _NC * _NS

_PITCH = 129


@functools.lru_cache(maxsize=None)
def _build(n_chunks, D, n_out):
    cpw = n_chunks // _NW

    mesh = plsc.VectorSubcoreMesh(core_axis_name="c", subcore_axis_name="s")

    @functools.partial(
        pl.kernel,
        mesh=mesh,
        out_type=jax.ShapeDtypeStruct((n_out // 128, 128), jnp.float32),
        scratch_types=[
            pltpu.VMEM((cpw, 128), jnp.int32),
            pltpu.VMEM((2, 128, D), jnp.float32),
            pltpu.VMEM((2, D, _PITCH), jnp.float32),
            pltpu.SemaphoreType.DMA,
            pltpu.SemaphoreType.DMA,
            pltpu.SemaphoreType.DMA,
            pltpu.SemaphoreType.DMA,
        ],
        compiler_params=pltpu.CompilerParams(
            use_tc_tiling_on_sc=False, needs_layout_passes=False),
    )
    def sc_gather(table_hbm, idx_hbm, out_hbm, idx_v, g_v, t_v,
                  g0sem, g1sem, s0sem, s1sem):
        wid = lax.axis_index("s") * _NC + lax.axis_index("c")
        chunk0 = wid * cpw
        pltpu.sync_copy(idx_hbm.at[pl.ds(chunk0, cpw)], idx_v)

        iota16 = lax.iota(jnp.int32, 16)
        gsems = (g0sem, g1sem)
        ssems = (s0sem, s1sem)

        def gather(g, buf):
            pltpu.async_copy(table_hbm.at[idx_v.at[g]], g_v.at[buf],
                             gsems[buf])

        def wait_gather(g, buf):
            pltpu.make_async_copy(table_hbm.at[idx_v.at[g]], g_v.at[buf],
                                  gsems[buf]).wait()

        def transpose(buf):
            g_ref = g_v.at[buf]
            t_ref = t_v.at[buf]

            @plsc.parallel_loop(0, 128, 1, unroll=16)
            def tok(j):
                cols = iota16 * 0 + j
                for c0 in range(0, D, 16):
                    v = g_ref[j, pl.ds(c0, 16)]
                    plsc.store_scatter(t_ref, [iota16 + c0, cols], v)

        def store(g, buf):
            rho = chunk0 + g
            d1 = rho // 128
            e = rho % 128
            base = d1 * (128 * D) + e * 8
            for a in range(D // 8):
                pltpu.async_copy(
                    t_v.at[buf, pl.ds(a * 8, 8), pl.ds(0, 128)],
                    out_hbm.at[pl.ds(base + a * 1024, 8)],
                    ssems[buf])

        def drain_store(buf):
            for a in range(D // 8):
                pltpu.make_async_copy(
                    t_v.at[buf, pl.ds(a * 8, 8), pl.ds(0, 128)],
                    out_hbm.at[pl.ds(a * 1024, 8)],
                    ssems[buf]).wait()

        def half(p, g, buf):
            wait_gather(g, buf)
            pl.when(p > 0)(lambda: drain_store(buf))
            transpose(buf)
            store(g, buf)
            @pl.when(g + 2 < cpw)
            def _():
                gather(g + 2, buf)

        gather(0, 0)
        gather(1, 1)

        def pair(p, carry):
            g0 = p * 2
            half(p, g0, 0)
            half(p, g0 + 1, 1)
            return carry

        lax.fori_loop(0, cpw // 2, pair, 0)
        drain_store(0)
        drain_store(1)

    return sc_gather


def kernel(x, tok_embed):
    s0, s1 = x.shape
    V, D = tok_embed.shape
    n_chunks = s0 * s1 // 128
    idx_t = x.T.reshape(n_chunks, 128).astype(jnp.int32)
    n_out = s0 * s1 * D
    o = _build(n_chunks, D, n_out)(tok_embed, idx_t)
    o5 = o.reshape(s1, D // 8, s0 // 128, 8, 128)
    return o5.transpose(2, 4, 0, 1, 3).reshape(s0, s1, D)


# speedup vs baseline: 1.2263x; 1.0072x over previous
import functools

import jax
import jax.numpy as jnp
from jax import lax
from jax.experimental import pallas as pl
from jax.experimental.pallas import tpu as pltpu
from jax.experimental.pallas import tpu_sc as plsc

_NC = 2
_NS = 16
_NW = _NC * _NS

_PITCH = 129


@functools.lru_cache(maxsize=None)
def _build(n_chunks, D, n_out):
    cpw = n_chunks // _NW

    mesh = plsc.VectorSubcoreMesh(core_axis_name="c", subcore_axis_name="s")

    @functools.partial(
        pl.kernel,
        mesh=mesh,
        out_type=jax.ShapeDtypeStruct((n_out // 128, 128), jnp.float32),
        scratch_types=[
            pltpu.VMEM((cpw, 128), jnp.int32),
            pltpu.VMEM((2, 128, D), jnp.float32),
            pltpu.VMEM((2, D, _PITCH), jnp.float32),
            pltpu.SemaphoreType.DMA,
            pltpu.SemaphoreType.DMA,
            pltpu.SemaphoreType.DMA,
            pltpu.SemaphoreType.DMA,
        ],
        compiler_params=pltpu.CompilerParams(
            use_tc_tiling_on_sc=False, needs_layout_passes=False),
    )
    def sc_gather(table_hbm, idx_hbm, out_hbm, idx_v, g_v, t_v,
                  g0sem, g1sem, s0sem, s1sem):
        wid = lax.axis_index("s") * _NC + lax.axis_index("c")
        chunk0 = wid * cpw
        pltpu.sync_copy(idx_hbm.at[pl.ds(chunk0, cpw)], idx_v)

        iota16 = lax.iota(jnp.int32, 16)
        gsems = (g0sem, g1sem)
        ssems = (s0sem, s1sem)

        def gather(g, buf):
            pltpu.async_copy(table_hbm.at[idx_v.at[g]], g_v.at[buf],
                             gsems[buf])

        def wait_gather(g, buf):
            pltpu.make_async_copy(table_hbm.at[idx_v.at[g]], g_v.at[buf],
                                  gsems[buf]).wait()

        def transpose(buf):
            g_ref = g_v.at[buf]
            t_ref = t_v.at[buf]

            @plsc.parallel_loop(0, 128, 1, unroll=8)
            def tok(j):
                cols = iota16 * 0 + j
                for c0 in range(0, D, 16):
                    v = g_ref[j, pl.ds(c0, 16)]
                    plsc.store_scatter(t_ref, [iota16 + c0, cols], v)

        def store(g, buf):
            rho = chunk0 + g
            d1 = rho // 128
            e = rho % 128
            base = d1 * (128 * D) + e * 8
            for a in range(D // 8):
                pltpu.async_copy(
                    t_v.at[buf, pl.ds(a * 8, 8), pl.ds(0, 128)],
                    out_hbm.at[pl.ds(base + a * 1024, 8)],
                    ssems[buf])

        def drain_store(buf):
            for a in range(D // 8):
                pltpu.make_async_copy(
                    t_v.at[buf, pl.ds(a * 8, 8), pl.ds(0, 128)],
                    out_hbm.at[pl.ds(a * 1024, 8)],
                    ssems[buf]).wait()

        def half(p, g, buf):
            wait_gather(g, buf)
            pl.when(p > 0)(lambda: drain_store(buf))
            transpose(buf)
            store(g, buf)
            @pl.when(g + 2 < cpw)
            def _():
                gather(g + 2, buf)

        gather(0, 0)
        gather(1, 1)

        def pair(p, carry):
            g0 = p * 2
            half(p, g0, 0)
            half(p, g0 + 1, 1)
            return carry

        lax.fori_loop(0, cpw // 2, pair, 0)
        drain_store(0)
        drain_store(1)

    return sc_gather


def kernel(x, tok_embed):
    s0, s1 = x.shape
    V, D = tok_embed.shape
    n_chunks = s0 * s1 // 128
    idx_t = x.T.reshape(n_chunks, 128).astype(jnp.int32)
    n_out = s0 * s1 * D
    o = _build(n_chunks, D, n_out)(tok_embed, idx_t)
    o5 = o.reshape(s1, D // 8, s0 // 128, 8, 128)
    return o5.transpose(2, 4, 0, 1, 3).reshape(s0, s1, D)
